# baseline (speedup 1.0000x reference)
import functools

import jax
import jax.numpy as jnp
from jax import lax
from jax.experimental import pallas as pl
from jax.experimental.pallas import tpu as pltpu
from jax.experimental.pallas import tpu_sc as plsc

_N = 81920
_C = 1000
_NC = 2
_NS = 16
_NW = _NC * _NS
_PER_W = _N // _NW
_CHUNK = 128
_ROWS = _PER_W // _CHUNK
_ROWS_TOTAL = _N // _CHUNK


def _ganloss_body(prob_hbm, tgt_hbm, rwd_hbm, out_hbm,
                  tgt_v, rwd_v, idx_v, gath_v, acc_v, sem):
    c = lax.axis_index("c")
    s = lax.axis_index("s")
    wid = s * _NC + c
    row0 = wid * _ROWS

    pltpu.sync_copy(tgt_hbm.at[wid], tgt_v)
    pltpu.sync_copy(rwd_hbm.at[wid], rwd_v)

    lanes = lax.iota(jnp.int32, 16)
    base_elem = row0 * _CHUNK

    for j in range(_ROWS):
        for k in range(_CHUNK // 16):
            t = tgt_v[j, pl.ds(k * 16, 16)]
            pos = base_elem + j * _CHUNK + k * 16 + lanes
            idx_v[j, pl.ds(k * 16, 16)] = pos * _C + t
        pltpu.make_async_copy(prob_hbm.at[idx_v.at[j]], gath_v.at[j],
                              sem).start()
    for j in range(_ROWS):
        pltpu.make_async_copy(prob_hbm.at[idx_v.at[j]], gath_v.at[j],
                              sem).wait()

    def body(j, acc):
        for k in range(_CHUNK // 16):
            g = gath_v[j, pl.ds(k * 16, 16)]
            r = rwd_v[j, pl.ds(k * 16, 16)]
            acc = acc - g * r
        return acc

    acc = lax.fori_loop(0, _ROWS, body, jnp.zeros((16,), jnp.float32))
    acc_v[...] = acc
    pltpu.sync_copy(acc_v, out_hbm.at[c, s])


@jax.jit
def _ganloss_sc(prob_flat, tgt2, rwd2):
    f = pl.kernel(
        _ganloss_body,
        out_type=jax.ShapeDtypeStruct((_NC, _NS, 16), jnp.float32),
        mesh=plsc.VectorSubcoreMesh(core_axis_name="c", subcore_axis_name="s"),
        scratch_types=[
            pltpu.VMEM((_ROWS, _CHUNK), jnp.int32),
            pltpu.VMEM((_ROWS, _CHUNK), jnp.float32),
            pltpu.VMEM((_ROWS, _CHUNK), jnp.int32),
            pltpu.VMEM((_ROWS, _CHUNK), jnp.float32),
            pltpu.VMEM((16,), jnp.float32),
            pltpu.SemaphoreType.DMA,
        ],
    )
    return f(prob_flat, tgt2, rwd2)


def kernel(prob, target, reward):
    prob_flat = prob.reshape(-1)
    tgt3 = target.astype(jnp.int32).reshape(_NW, _ROWS, _CHUNK)
    rwd3 = reward.reshape(_NW, _ROWS, _CHUNK)
    partials = _ganloss_sc(prob_flat, tgt3, rwd3)
    return jnp.sum(partials)


# speedup vs baseline: 1.6299x; 1.6299x over previous
import jax
import jax.numpy as jnp
from jax import lax
from jax._src.state import indexing as _state_indexing
from jax._src.state import types as _state_types
from jax.experimental import pallas as pl
from jax.experimental.pallas import tpu as pltpu
from jax.experimental.pallas import tpu_sc as plsc

_N = 81920
_C = 1000
_WIN = 128
_NP = 8

_NC = 2
_NS = 16
_NW = _NC * _NS
_PER_W = _N // _NW
_CHUNK = 128
_BCAP = _PER_W + _CHUNK


def _ganloss_body(prob_hbm, tgt_hbm, rwd_hbm, out_hbm,
                  tgt_v, rwd_v, bkt_v, gath_v, acc_v, sem):
    c = lax.axis_index("c")
    s = lax.axis_index("s")
    wid = s * _NC + c
    base = wid * _PER_W

    pltpu.sync_copy(tgt_hbm.at[pl.ds(base, _PER_W)], tgt_v)
    pltpu.sync_copy(rwd_hbm.at[pl.ds(base, _PER_W)], rwd_v)

    lanes = lax.iota(jnp.int32, 16)

    def part_body(jj, curs):
        t = tgt_v[pl.ds(jj * 16, 16)]
        iv = base + jj * 16 + lanes
        g = t >> 7
        dest = jnp.zeros((16,), jnp.int32)
        new_curs = []
        for b in range(_NP):
            m = (g == b).astype(jnp.int32)
            excl = plsc.cumsum(m) - m
            dest = dest + m * (b * _BCAP + curs[b] + excl)
            new_curs.append(curs[b] + jnp.sum(m))
        plsc.store_scatter(bkt_v, [dest], iv)
        return tuple(new_curs)

    cnts = lax.fori_loop(0, _PER_W // 16, part_body, (jnp.int32(0),) * _NP)

    basev = jnp.zeros((16,), jnp.int32) + base
    for b in range(_NP):
        for k in range(_CHUNK // 16):
            bkt_v[pl.ds(b * _BCAP + cnts[b] + k * 16, 16)] = basev

    acc = jnp.zeros((16,), jnp.float32)
    for b in range(_NP):
        nch = (cnts[b] + _CHUNK - 1) // _CHUNK
        woff = b * _WIN

        def chunk_body(ch, acc, b=b, woff=woff, cnt=cnts[b]):
            st = b * _BCAP + ch * _CHUNK
            nd = _state_indexing.NDIndexer(
                (bkt_v.at[pl.ds(st, _CHUNK)],
                 _state_indexing.Slice(woff, _WIN)),
                (_N, _C), (_CHUNK,))
            cp = pltpu.make_async_copy(
                _state_types.TransformedRef(prob_hbm, (nd,)),
                gath_v, sem)
            cp.start()
            cp.wait()
            for k in range(_CHUNK // 16):
                ids = bkt_v[pl.ds(st + k * 16, 16)]
                loc = ids - base
                t = plsc.load_gather(tgt_v, [loc])
                off = t & 127
                rows = k * 16 + lanes
                val = plsc.load_gather(gath_v, [rows, off])
                r = plsc.load_gather(rwd_v, [loc])
                valid = (ch * _CHUNK + k * 16 + lanes) < cnt
                acc = acc - jnp.where(valid, val * r, 0.0)
            return acc

        acc = lax.fori_loop(0, nch, chunk_body, acc)

    acc_v[...] = acc
    pltpu.sync_copy(acc_v, out_hbm.at[c, s])


@jax.jit
def _ganloss_sc(prob, tgt, rwd):
    f = pl.kernel(
        _ganloss_body,
        out_type=jax.ShapeDtypeStruct((_NC, _NS, 16), jnp.float32),
        mesh=plsc.VectorSubcoreMesh(core_axis_name="c", subcore_axis_name="s"),
        compiler_params=pltpu.CompilerParams(needs_layout_passes=False),
        scratch_types=[
            pltpu.VMEM((_PER_W,), jnp.int32),
            pltpu.VMEM((_PER_W,), jnp.float32),
            pltpu.VMEM((_NP * _BCAP,), jnp.int32),
            pltpu.VMEM((_CHUNK, _WIN), jnp.float32),
            pltpu.VMEM((16,), jnp.float32),
            pltpu.SemaphoreType.DMA,
        ],
    )
    return f(prob, tgt, rwd)


def kernel(prob, target, reward):
    partials = _ganloss_sc(prob, target.astype(jnp.int32), reward)
    return jnp.sum(partials)


# speedup vs baseline: 1.7080x; 1.0479x over previous
import jax
import jax.numpy as jnp
from jax import lax
from jax._src.state import indexing as _state_indexing
from jax._src.state import types as _state_types
from jax.experimental import pallas as pl
from jax.experimental.pallas import tpu as pltpu
from jax.experimental.pallas import tpu_sc as plsc

_N = 81920
_C = 1000
_WIN = 128
_NP = 8

_NC = 2
_NS = 16
_NW = _NC * _NS
_PER_W = _N // _NW
_CHUNK = 128
_GRP = 2
_BCAP = _PER_W + _GRP * _CHUNK


def _ganloss_body(prob_hbm, tgt_hbm, rwd_hbm, out_hbm,
                  tgt_v, rwd_v, bkt_v, gath_v, acc_v, sem):
    c = lax.axis_index("c")
    s = lax.axis_index("s")
    wid = s * _NC + c
    base = wid * _PER_W

    pltpu.sync_copy(tgt_hbm.at[pl.ds(base, _PER_W)], tgt_v)
    pltpu.sync_copy(rwd_hbm.at[pl.ds(base, _PER_W)], rwd_v)

    lanes = lax.iota(jnp.int32, 16)

    for b in range(_NP):
        pltpu.sync_copy(tgt_hbm.at[pl.ds(0, _BCAP)],
                        bkt_v.at[pl.ds(b * _BCAP, _BCAP)])

    def part_body(jj, curs):
        t = tgt_v[pl.ds(jj * 16, 16)]
        iv = base + jj * 16 + lanes
        g = t >> 7
        dest = jnp.zeros((16,), jnp.int32)
        new_curs = []
        for b in range(_NP):
            m = (g == b).astype(jnp.int32)
            excl = plsc.cumsum(m) - m
            dest = dest + m * (b * _BCAP + curs[b] + excl)
            new_curs.append(curs[b] + jnp.sum(m))
        plsc.store_scatter(bkt_v, [dest], iv)
        return tuple(new_curs)

    cnts = lax.fori_loop(0, _PER_W // 16, part_body, (jnp.int32(0),) * _NP)

    acc = jnp.zeros((16,), jnp.float32)
    for b in range(_NP):
        ngr = (cnts[b] + _GRP * _CHUNK - 1) // (_GRP * _CHUNK)
        woff = b * _WIN

        def grp_body(gr, acc, b=b, woff=woff, cnt=cnts[b]):
            st0 = b * _BCAP + gr * (_GRP * _CHUNK)

            def _copy(q):
                nd = _state_indexing.NDIndexer(
                    (bkt_v.at[pl.ds(st0 + q * _CHUNK, _CHUNK)],
                     _state_indexing.Slice(woff, _WIN)),
                    (_N, _C), (_CHUNK,))
                return pltpu.make_async_copy(
                    _state_types.TransformedRef(prob_hbm, (nd,)),
                    gath_v.at[q], sem)

            for q in range(_GRP):
                _copy(q).start()
            for q in range(_GRP):
                _copy(q).wait()

            def ext(qk, a):
                q = qk >> 3
                k = qk & 7
                ids = bkt_v[pl.ds(st0 + q * _CHUNK + k * 16, 16)]
                loc = jnp.clip(ids - base, 0, _PER_W - 1)
                t = plsc.load_gather(tgt_v, [loc])
                off = t & 127
                rows = k * 16 + lanes
                val = plsc.load_gather(gath_v, [jnp.zeros((16,), jnp.int32)
                                                + q, rows, off])
                valid = (gr * _GRP * _CHUNK + q * _CHUNK
                         + k * 16 + lanes) < cnt
                r = plsc.load_gather(rwd_v, [loc])
                return a - jnp.where(valid, val * r, 0.0)

            return lax.fori_loop(0, _GRP * (_CHUNK // 16), ext, acc)

        acc = lax.fori_loop(0, ngr, grp_body, acc)

    acc_v[...] = acc
    pltpu.sync_copy(acc_v, out_hbm.at[c, s])


@jax.jit
def _ganloss_sc(prob, tgt, rwd):
    f = pl.kernel(
        _ganloss_body,
        out_type=jax.ShapeDtypeStruct((_NC, _NS, 16), jnp.float32),
        mesh=plsc.VectorSubcoreMesh(core_axis_name="c", subcore_axis_name="s"),
        compiler_params=pltpu.CompilerParams(needs_layout_passes=False),
        scratch_types=[
            pltpu.VMEM((_PER_W,), jnp.int32),
            pltpu.VMEM((_PER_W,), jnp.float32),
            pltpu.VMEM((_NP * _BCAP,), jnp.int32),
            pltpu.VMEM((_GRP, _CHUNK, _WIN), jnp.float32),
            pltpu.VMEM((16,), jnp.float32),
            pltpu.SemaphoreType.DMA,
        ],
    )
    return f(prob, tgt, rwd)


def kernel(prob, target, reward):
    partials = _ganloss_sc(prob, target.astype(jnp.int32), reward)
    return jnp.sum(partials)
